# baseline (speedup 1.0000x reference)
import functools

import jax
import jax.numpy as jnp
from jax.experimental import pallas as pl
from jax.experimental.pallas import tpu as pltpu

_SUB = 8192


def _flash_body(num_tiles, n_sub, q_ref, m_ref, o_ref, acc_ref, mx_ref, l_ref):
    i = pl.program_id(0)

    @pl.when(i == 0)
    def _init():
        acc_ref[...] = jnp.zeros_like(acc_ref)
        mx_ref[...] = jnp.full_like(mx_ref, -1e30)
        l_ref[...] = jnp.zeros_like(l_ref)

    q = q_ref[...]

    def score(k):
        m_k = m_ref[pl.ds(k * _SUB, _SUB), :]
        s_k = jax.lax.dot_general(
            q, m_k, (((1,), (1,)), ((), ())),
            preferred_element_type=jnp.float32,
        )
        return m_k, s_k

    def consume(state, m_k, s_k):
        mx, l, acc = state
        mx_new = jnp.maximum(mx, jnp.max(s_k, axis=1, keepdims=True))
        corr = jnp.exp(mx - mx_new)
        p = jnp.exp(s_k - mx_new)
        l_new = l * corr + jnp.sum(p, axis=1, keepdims=True)
        acc_new = acc * corr + jnp.dot(
            p, m_k, preferred_element_type=jnp.float32
        )
        return mx_new, l_new, acc_new

    state = (mx_ref[...], l_ref[...], acc_ref[...])
    pending = score(0)
    for k in range(1, n_sub):
        nxt = score(k)
        state = consume(state, *pending)
        pending = nxt
    state = consume(state, *pending)

    mx, l, acc = state
    mx_ref[...] = mx
    l_ref[...] = l
    acc_ref[...] = acc

    @pl.when(i == num_tiles - 1)
    def _finish():
        o_ref[...] = acc / l


@functools.partial(jax.jit, static_argnames=("tile",))
def _content_addressed_read(query, memory, tile=32768):
    batch, dim = query.shape
    num_slots = memory.shape[0]
    num_tiles = num_slots // tile
    n_sub = tile // _SUB

    return pl.pallas_call(
        functools.partial(_flash_body, num_tiles, n_sub),
        grid=(num_tiles,),
        in_specs=[
            pl.BlockSpec((batch, dim), lambda i: (0, 0)),
            pl.BlockSpec((tile, dim), lambda i: (i, 0)),
        ],
        out_specs=pl.BlockSpec((batch, dim), lambda i: (0, 0)),
        out_shape=jax.ShapeDtypeStruct((batch, dim), jnp.float32),
        scratch_shapes=[
            pltpu.VMEM((batch, dim), jnp.float32),
            pltpu.VMEM((batch, 1), jnp.float32),
            pltpu.VMEM((batch, 1), jnp.float32),
        ],
    )(query, memory)


def kernel(query, memory):
    return _content_addressed_read(query, memory)


# speedup vs baseline: 1.0457x; 1.0457x over previous
import functools

import jax
import jax.numpy as jnp
from jax.experimental import pallas as pl
from jax.experimental.pallas import tpu as pltpu

_SUB = 8192


def _flash_body(num_tiles, n_sub, q_ref, m_ref, o_ref, acc_ref, mx_ref, l_ref):
    i = pl.program_id(0)

    @pl.when(i == 0)
    def _init():
        acc_ref[...] = jnp.zeros_like(acc_ref)
        mx_ref[...] = jnp.full_like(mx_ref, -1e30)
        l_ref[...] = jnp.zeros_like(l_ref)

    q = q_ref[...]

    def score(k):
        m_k = m_ref[pl.ds(k * _SUB, _SUB), :]
        s_k = jax.lax.dot_general(
            q, m_k, (((1,), (1,)), ((), ())),
            preferred_element_type=jnp.float32,
        )
        return m_k, s_k

    def consume(state, m_k, s_k):
        mx, l, acc = state
        mx_new = jnp.maximum(mx, jnp.max(s_k, axis=1, keepdims=True))
        corr = jnp.exp(mx - mx_new)
        p = jnp.exp(s_k - mx_new)
        l_new = l * corr + jnp.sum(p, axis=1, keepdims=True)
        acc_new = acc * corr + jnp.dot(
            p, m_k, preferred_element_type=jnp.float32
        )
        return mx_new, l_new, acc_new

    state = (mx_ref[...], l_ref[...], acc_ref[...])
    pending = score(0)
    for k in range(1, n_sub):
        nxt = score(k)
        state = consume(state, *pending)
        pending = nxt
    state = consume(state, *pending)

    mx, l, acc = state
    mx_ref[...] = mx
    l_ref[...] = l
    acc_ref[...] = acc

    @pl.when(i == num_tiles - 1)
    def _finish():
        o_ref[...] = acc / l


@functools.partial(jax.jit, static_argnames=("tile",))
def _content_addressed_read(query, memory, tile=16384):
    batch, dim = query.shape
    num_slots = memory.shape[0]
    num_tiles = num_slots // tile
    n_sub = tile // _SUB

    return pl.pallas_call(
        functools.partial(_flash_body, num_tiles, n_sub),
        grid=(num_tiles,),
        in_specs=[
            pl.BlockSpec((batch, dim), lambda i: (0, 0)),
            pl.BlockSpec((tile, dim), lambda i: (i, 0)),
        ],
        out_specs=pl.BlockSpec((batch, dim), lambda i: (0, 0)),
        out_shape=jax.ShapeDtypeStruct((batch, dim), jnp.float32),
        scratch_shapes=[
            pltpu.VMEM((batch, dim), jnp.float32),
            pltpu.VMEM((batch, 1), jnp.float32),
            pltpu.VMEM((batch, 1), jnp.float32),
        ],
    )(query, memory)


def kernel(query, memory):
    return _content_addressed_read(query, memory)


# speedup vs baseline: 1.0622x; 1.0158x over previous
import functools

import jax
import jax.numpy as jnp
from jax.experimental import pallas as pl
from jax.experimental.pallas import tpu as pltpu

_SUB = 4096


def _flash_body(num_tiles, n_sub, q_ref, m_ref, o_ref, acc_ref, mx_ref, l_ref):
    i = pl.program_id(0)

    @pl.when(i == 0)
    def _init():
        acc_ref[...] = jnp.zeros_like(acc_ref)
        mx_ref[...] = jnp.full_like(mx_ref, -1e30)
        l_ref[...] = jnp.zeros_like(l_ref)

    q = q_ref[...]

    def score(k):
        m_k = m_ref[pl.ds(k * _SUB, _SUB), :]
        s_k = jax.lax.dot_general(
            q, m_k, (((1,), (1,)), ((), ())),
            preferred_element_type=jnp.float32,
        )
        return m_k, s_k

    def consume(state, m_k, s_k):
        mx, l, acc = state
        mx_new = jnp.maximum(mx, jnp.max(s_k, axis=1, keepdims=True))
        corr = jnp.exp(mx - mx_new)
        p = jnp.exp(s_k - mx_new)
        l_new = l * corr + jnp.sum(p, axis=1, keepdims=True)
        acc_new = acc * corr + jnp.dot(
            p, m_k, preferred_element_type=jnp.float32
        )
        return mx_new, l_new, acc_new

    state = (mx_ref[...], l_ref[...], acc_ref[...])
    pend = [score(k) for k in range(min(2, n_sub))]
    for k in range(2, n_sub):
        pend.append(score(k))
        state = consume(state, *pend.pop(0))
    while pend:
        state = consume(state, *pend.pop(0))

    mx, l, acc = state
    mx_ref[...] = mx
    l_ref[...] = l
    acc_ref[...] = acc

    @pl.when(i == num_tiles - 1)
    def _finish():
        o_ref[...] = acc / l


@functools.partial(jax.jit, static_argnames=("tile",))
def _content_addressed_read(query, memory, tile=16384):
    batch, dim = query.shape
    num_slots = memory.shape[0]
    num_tiles = num_slots // tile
    n_sub = tile // _SUB

    return pl.pallas_call(
        functools.partial(_flash_body, num_tiles, n_sub),
        grid=(num_tiles,),
        in_specs=[
            pl.BlockSpec((batch, dim), lambda i: (0, 0)),
            pl.BlockSpec((tile, dim), lambda i: (i, 0)),
        ],
        out_specs=pl.BlockSpec((batch, dim), lambda i: (0, 0)),
        out_shape=jax.ShapeDtypeStruct((batch, dim), jnp.float32),
        scratch_shapes=[
            pltpu.VMEM((batch, dim), jnp.float32),
            pltpu.VMEM((batch, 1), jnp.float32),
            pltpu.VMEM((batch, 1), jnp.float32),
        ],
    )(query, memory)


def kernel(query, memory):
    return _content_addressed_read(query, memory)
